# baseline (speedup 1.0000x reference)
import functools

import jax
import jax.numpy as jnp
from jax import lax
from jax.experimental import pallas as pl
from jax.experimental.pallas import tpu as pltpu
from jax.experimental.pallas import tpu_sc as plsc

_NUM_CORES = 2
_NUM_SUBCORES = 16


def _argmax_body(n_total, x_ref, out_ref, m_ref, a_ref):
    i = pl.program_id(0)
    nc = pl.num_programs(0)
    x = x_ref[...]
    s, b, c = x.shape
    gidx = i * c + lax.broadcasted_iota(jnp.int32, x.shape, 2)
    cmax = jnp.max(x, axis=-1)
    masked = jnp.where(x == cmax[..., None], gidx, jnp.int32(2**30))
    carg = jnp.min(masked, axis=-1)

    @pl.when(i == 0)
    def _():
        m_ref[...] = cmax
        a_ref[...] = carg

    @pl.when(i > 0)
    def _():
        upd = cmax > m_ref[...]
        m_ref[...] = jnp.where(upd, cmax, m_ref[...])
        a_ref[...] = jnp.where(upd, carg, a_ref[...])

    @pl.when(i == nc - 1)
    def _():
        a_t = a_ref[...].T
        row = lax.broadcasted_iota(jnp.int32, a_t.shape, 0)
        out_ref[...] = a_t + n_total * row


def _flat_argmax_indices(samples, chunk=4096):
    s, b, n = samples.shape
    grid = n // chunk
    return pl.pallas_call(
        functools.partial(_argmax_body, n),
        grid=(grid,),
        in_specs=[pl.BlockSpec((s, b, chunk), lambda i: (0, 0, i))],
        out_specs=pl.BlockSpec((b, s), lambda i: (0, 0)),
        out_shape=jax.ShapeDtypeStruct((b, s), jnp.int32),
        scratch_shapes=[
            pltpu.VMEM((s, b), jnp.float32),
            pltpu.VMEM((s, b), jnp.int32),
        ],
    )(samples)


def _sc_gather(table, idx, d):
    r = idx.shape[0]
    n_workers = 16
    rows_per_w = r // n_workers
    mesh = plsc.VectorSubcoreMesh(core_axis_name="c", subcore_axis_name="s")

    @functools.partial(
        pl.kernel,
        mesh=mesh,
        out_type=jax.ShapeDtypeStruct((r, d), jnp.float32),
        scratch_types=[
            pltpu.VMEM((rows_per_w,), jnp.int32),
            pltpu.VMEM((rows_per_w, d), jnp.float32),
            pltpu.SemaphoreType.DMA,
        ],
        compiler_params=pltpu.CompilerParams(use_tc_tiling_on_sc=False),
    )
    def gather_kernel(table_hbm, idx_hbm, out_hbm, idx_v, rows_v, sem):
        wid = lax.axis_index("s") * _NUM_CORES + lax.axis_index("c")

        @pl.when(wid < n_workers)
        def _():
            base = wid * rows_per_w
            pltpu.sync_copy(idx_hbm.at[pl.ds(base, rows_per_w)], idx_v)
            pltpu.async_copy(table_hbm.at[idx_v], rows_v, sem).wait()
            pltpu.sync_copy(rows_v, out_hbm.at[pl.ds(base, rows_per_w)])

    return gather_kernel(table, idx)


def kernel(X, samples, num_samples):
    b, n, d = X.shape
    s = samples.shape[0]
    idx = _flat_argmax_indices(samples)
    out = _sc_gather(X.reshape(b * n, d), idx.reshape(b * s), d)
    return out.reshape(b, s, d)


# speedup vs baseline: 9.8515x; 9.8515x over previous
import functools

import jax
import jax.numpy as jnp
from jax import lax
from jax.experimental import pallas as pl
from jax.experimental.pallas import tpu as pltpu
from jax.experimental.pallas import tpu_sc as plsc

_NUM_CORES = 2
_NUM_SUBCORES = 16
_LANES = 16


def _argmax_body(x_ref, out_ref, m_ref, a_ref):
    i = pl.program_id(0)
    nc = pl.num_programs(0)
    x = x_ref[...]
    s, b, c = x.shape
    gidx = i * c + lax.broadcasted_iota(jnp.int32, x.shape, 2)
    cmax = jnp.max(x, axis=-1)
    masked = jnp.where(x == cmax[..., None], gidx, jnp.int32(2**30))
    carg = jnp.min(masked, axis=-1)

    @pl.when(i == 0)
    def _():
        m_ref[...] = cmax
        a_ref[...] = carg

    @pl.when(i > 0)
    def _():
        upd = cmax > m_ref[...]
        m_ref[...] = jnp.where(upd, cmax, m_ref[...])
        a_ref[...] = jnp.where(upd, carg, a_ref[...])

    @pl.when(i == nc - 1)
    def _():
        pad = jnp.zeros((s, 128 - b), jnp.int32)
        out_ref[...] = jnp.concatenate([a_ref[...], pad], axis=1)


def _argmax_indices(samples, chunk=4096):
    s, b, n = samples.shape
    grid = n // chunk
    return pl.pallas_call(
        _argmax_body,
        grid=(grid,),
        in_specs=[pl.BlockSpec((s, b, chunk), lambda i: (0, 0, i))],
        out_specs=pl.BlockSpec((s, 128), lambda i: (0, 0)),
        out_shape=jax.ShapeDtypeStruct((s, 128), jnp.int32),
        scratch_shapes=[
            pltpu.VMEM((s, b), jnp.float32),
            pltpu.VMEM((s, b), jnp.int32),
        ],
    )(samples)


def _sc_gather(x5, idx, s_sz, b_sz, d):
    n_workers = _NUM_CORES * _NUM_SUBCORES
    b_per_w = b_sz * s_sz // n_workers
    mesh = plsc.VectorSubcoreMesh(core_axis_name="c", subcore_axis_name="s")

    @functools.partial(
        pl.kernel,
        mesh=mesh,
        compiler_params=pltpu.CompilerParams(needs_layout_passes=False),
        out_type=jax.ShapeDtypeStruct((b_sz, s_sz, d), jnp.float32),
        scratch_types=[
            pltpu.VMEM((_LANES,), jnp.int32),
            pltpu.VMEM((d // 8, 8, 128), jnp.float32),
            pltpu.VMEM((d,), jnp.float32),
        ],
    )
    def gather_kernel(x5_hbm, idx_hbm, out_hbm, idx_v, win_v, col_v):
        wid = lax.axis_index("s") * _NUM_CORES + lax.axis_index("c")
        s_idx = wid // b_per_w
        quarter = wid % b_per_w
        pltpu.sync_copy(idx_hbm.at[s_idx, pl.ds(0, _LANES)], idx_v)
        for j in range(b_per_w):
            b_idx = quarter * b_per_w + j
            n_vec = plsc.load_gather(idx_v, [jnp.full((_LANES,), b_idx, jnp.int32)])
            n_hi = n_vec[0] >> 7
            pltpu.sync_copy(x5_hbm.at[b_idx, :, n_hi, :, :], win_v)
            lanes = lax.broadcasted_iota(jnp.int32, (_LANES,), 0)
            for g in range(d // _LANES):
                rows = g * _LANES + lanes
                col_v[pl.ds(g * _LANES, _LANES)] = plsc.load_gather(
                    win_v, [rows >> 3, rows & 7, n_vec & 127]
                )
            pltpu.sync_copy(col_v, out_hbm.at[b_idx, s_idx])

    return gather_kernel(x5, idx)


def kernel(X, samples, num_samples):
    b, n, d = X.shape
    s = samples.shape[0]
    idx = _argmax_indices(samples)
    x5 = (
        X.transpose(0, 2, 1)
        .reshape(b, d // 8, 8, n // 128, 128)
        .transpose(0, 1, 3, 2, 4)
    )
    return _sc_gather(x5, idx, s, b, d)


# speedup vs baseline: 10.6024x; 1.0762x over previous
import functools

import jax
import jax.numpy as jnp
from jax import lax
from jax.experimental import pallas as pl
from jax.experimental.pallas import tpu as pltpu
from jax.experimental.pallas import tpu_sc as plsc

_NUM_CORES = 2
_NUM_SUBCORES = 16
_LANES = 16


def _argmax_body(x_ref, out_ref, m_ref, a_ref):
    i = pl.program_id(0)
    nc = pl.num_programs(0)
    x = x_ref[...]
    s, b, c = x.shape
    gidx = i * c + lax.broadcasted_iota(jnp.int32, x.shape, 2)
    cmax = jnp.max(x, axis=-1)
    masked = jnp.where(x == cmax[..., None], gidx, jnp.int32(2**30))
    carg = jnp.min(masked, axis=-1)

    @pl.when(i == 0)
    def _():
        m_ref[...] = cmax
        a_ref[...] = carg

    @pl.when(i > 0)
    def _():
        upd = cmax > m_ref[...]
        m_ref[...] = jnp.where(upd, cmax, m_ref[...])
        a_ref[...] = jnp.where(upd, carg, a_ref[...])

    @pl.when(i == nc - 1)
    def _():
        pad = jnp.zeros((s, 128 - b), jnp.int32)
        out_ref[...] = jnp.concatenate([a_ref[...], pad], axis=1)


def _argmax_indices(samples, chunk=4096):
    s, b, n = samples.shape
    grid = n // chunk
    return pl.pallas_call(
        _argmax_body,
        grid=(grid,),
        in_specs=[pl.BlockSpec((s, b, chunk), lambda i: (0, 0, i))],
        out_specs=pl.BlockSpec((s, 128), lambda i: (0, 0)),
        out_shape=jax.ShapeDtypeStruct((s, 128), jnp.int32),
        scratch_shapes=[
            pltpu.VMEM((s, b), jnp.float32),
            pltpu.VMEM((s, b), jnp.int32),
        ],
    )(samples)


def _sc_gather(x5, idx, s_sz, b_sz, d):
    n_workers = _NUM_CORES * _NUM_SUBCORES
    b_per_w = b_sz * s_sz // n_workers
    mesh = plsc.VectorSubcoreMesh(core_axis_name="c", subcore_axis_name="s")

    @functools.partial(
        pl.kernel,
        mesh=mesh,
        compiler_params=pltpu.CompilerParams(needs_layout_passes=False),
        out_type=jax.ShapeDtypeStruct((b_sz, s_sz, d), jnp.float32),
        scratch_types=[
            pltpu.VMEM((_LANES,), jnp.int32),
            pltpu.VMEM((b_per_w, d // 8, 8, 128), jnp.float32),
            pltpu.VMEM((b_per_w, d), jnp.float32),
            pltpu.SemaphoreType.DMA,
            pltpu.SemaphoreType.DMA,
        ],
    )
    def gather_kernel(x5_hbm, idx_hbm, out_hbm, idx_v, win_v, col_v, wsem, osem):
        wid = lax.axis_index("s") * _NUM_CORES + lax.axis_index("c")
        s_idx = wid // b_per_w
        quarter = wid % b_per_w
        pltpu.sync_copy(idx_hbm.at[s_idx, pl.ds(0, _LANES)], idx_v)
        n_vecs = []
        win_dmas = []
        for j in range(b_per_w):
            b_idx = quarter * b_per_w + j
            n_vec = plsc.load_gather(idx_v, [jnp.full((_LANES,), b_idx, jnp.int32)])
            n_hi = n_vec[0] >> 7
            n_vecs.append(n_vec)
            win_dmas.append(
                pltpu.async_copy(x5_hbm.at[b_idx, :, n_hi, :, :], win_v.at[j], wsem)
            )
        lanes = lax.broadcasted_iota(jnp.int32, (_LANES,), 0)
        out_dmas = []
        for j in range(b_per_w):
            b_idx = quarter * b_per_w + j
            win_dmas[j].wait()
            for g in range(d // _LANES):
                rows = g * _LANES + lanes
                col_v[j, pl.ds(g * _LANES, _LANES)] = plsc.load_gather(
                    win_v, [jnp.full((_LANES,), j, jnp.int32), rows >> 3, rows & 7, n_vecs[j] & 127]
                )
            out_dmas.append(
                pltpu.async_copy(col_v.at[j], out_hbm.at[b_idx, s_idx], osem)
            )
        for dma in out_dmas:
            dma.wait()

    return gather_kernel(x5, idx)


def kernel(X, samples, num_samples):
    b, n, d = X.shape
    s = samples.shape[0]
    idx = _argmax_indices(samples)
    x5 = (
        X.transpose(0, 2, 1)
        .reshape(b, d // 8, 8, n // 128, 128)
        .transpose(0, 1, 3, 2, 4)
    )
    return _sc_gather(x5, idx, s, b, d)


# speedup vs baseline: 10.8422x; 1.0226x over previous
import functools

import jax
import jax.numpy as jnp
from jax import lax
from jax.experimental import pallas as pl
from jax.experimental.pallas import tpu as pltpu
from jax.experimental.pallas import tpu_sc as plsc

_NUM_CORES = 2
_NUM_SUBCORES = 16
_LANES = 16


def _argmax_body(x_ref, out_ref, m_ref, a_ref):
    i = pl.program_id(0)
    nc = pl.num_programs(0)
    x = x_ref[...]
    s, b, c = x.shape
    gidx = i * c + lax.broadcasted_iota(jnp.int32, x.shape, 2)
    cmax = jnp.max(x, axis=-1)
    masked = jnp.where(x == cmax[..., None], gidx, jnp.int32(2**30))
    carg = jnp.min(masked, axis=-1)

    @pl.when(i == 0)
    def _():
        m_ref[...] = cmax
        a_ref[...] = carg

    @pl.when(i > 0)
    def _():
        upd = cmax > m_ref[...]
        m_ref[...] = jnp.where(upd, cmax, m_ref[...])
        a_ref[...] = jnp.where(upd, carg, a_ref[...])

    @pl.when(i == nc - 1)
    def _():
        pad = jnp.zeros((s, 128 - b), jnp.int32)
        out_ref[...] = jnp.concatenate([a_ref[...], pad], axis=1)


def _argmax_indices(samples, chunk=4096):
    s, b, n = samples.shape
    grid = n // chunk
    return pl.pallas_call(
        _argmax_body,
        grid=(grid,),
        in_specs=[pl.BlockSpec((s, b, chunk), lambda i: (0, 0, i))],
        out_specs=pl.BlockSpec((s, 128), lambda i: (0, 0)),
        out_shape=jax.ShapeDtypeStruct((s, 128), jnp.int32),
        scratch_shapes=[
            pltpu.VMEM((s, b), jnp.float32),
            pltpu.VMEM((s, b), jnp.int32),
        ],
    )(samples)


def _sc_gather(x5, idx, s_sz, b_sz, d):
    n_workers = _NUM_SUBCORES
    b_per_w = b_sz * s_sz // n_workers
    mesh = plsc.VectorSubcoreMesh(
        core_axis_name="c", subcore_axis_name="s", num_cores=1
    )

    @functools.partial(
        pl.kernel,
        mesh=mesh,
        compiler_params=pltpu.CompilerParams(needs_layout_passes=False),
        out_type=jax.ShapeDtypeStruct((b_sz, s_sz, d), jnp.float32),
        scratch_types=[
            pltpu.VMEM((_LANES,), jnp.int32),
            pltpu.VMEM((b_per_w, d // 8, 8, 128), jnp.float32),
            pltpu.VMEM((b_per_w, d), jnp.float32),
            pltpu.SemaphoreType.DMA,
            pltpu.SemaphoreType.DMA,
        ],
    )
    def gather_kernel(x5_hbm, idx_hbm, out_hbm, idx_v, win_v, col_v, wsem, osem):
        wid = lax.axis_index("s")
        base = wid * b_per_w
        s_idx = base // b_sz
        pltpu.sync_copy(idx_hbm.at[s_idx, pl.ds(0, _LANES)], idx_v)
        n_vecs = []
        win_dmas = []
        for j in range(b_per_w):
            b_idx = base % b_sz + j
            n_vec = plsc.load_gather(idx_v, [jnp.full((_LANES,), b_idx, jnp.int32)])
            n_hi = n_vec[0] >> 7
            n_vecs.append(n_vec)
            win_dmas.append(
                pltpu.async_copy(x5_hbm.at[b_idx, :, n_hi, :, :], win_v.at[j], wsem)
            )
        lanes = lax.broadcasted_iota(jnp.int32, (_LANES,), 0)
        out_dmas = []
        for j in range(b_per_w):
            b_idx = base % b_sz + j
            win_dmas[j].wait()
            for g in range(d // _LANES):
                rows = g * _LANES + lanes
                col_v[j, pl.ds(g * _LANES, _LANES)] = plsc.load_gather(
                    win_v, [jnp.full((_LANES,), j, jnp.int32), rows >> 3, rows & 7, n_vecs[j] & 127]
                )
            out_dmas.append(
                pltpu.async_copy(col_v.at[j], out_hbm.at[b_idx, s_idx], osem)
            )
        for dma in out_dmas:
            dma.wait()

    return gather_kernel(x5, idx)


def kernel(X, samples, num_samples):
    b, n, d = X.shape
    s = samples.shape[0]
    idx = _argmax_indices(samples)
    x5 = (
        X.transpose(0, 2, 1)
        .reshape(b, d // 8, 8, n // 128, 128)
        .transpose(0, 1, 3, 2, 4)
    )
    return _sc_gather(x5, idx, s, b, d)
